# baseline (speedup 1.0000x reference)
import jax
import jax.numpy as jnp
from jax.experimental import pallas as pl
from jax.experimental.pallas import tpu as pltpu

_K = 40
_EPS = 1e-5
_NG = 8
_INF_BITS = 0x7F800000
_BIG = 3.0e38
_HI = jax.lax.Precision.HIGHEST


def _dot_t(a, b, precision=None):
    return jax.lax.dot_general(
        a, b, (((1,), (1,)), ((), ())),
        precision=precision, preferred_element_type=jnp.float32)


def _stage_a(x_ref, ws_ref, bs_ref, wh_ref, bh_ref, wo1_ref,
             s5_ref, h_ref, xo_ref):
    xb = x_ref[...]
    s = _dot_t(xb, ws_ref[...]) + bs_ref[...]
    s2 = jnp.sum(s * s, axis=1, keepdims=True)
    s5_ref[...] = jnp.concatenate([s, s2], axis=1)
    h_ref[...] = _dot_t(xb, wh_ref[...]) + bh_ref[...]
    xo_ref[...] = _dot_t(xb, wo1_ref[...])


_CC = 256


def _stage_b(bnds_ref, s5r_ref, bcol_ref, s5_ref, brow_ref, ht_ref, xo_ref,
             wo2_ref, bo2_ref, out0_ref, sum1_ref, ssq1_ref, bits_ref):
    i = pl.program_id(0)
    k_lo = bnds_ref[0, i]
    k_hi = bnds_ref[1, i]

    s5r = s5r_ref[...]
    sr = s5r[:, 0:4]
    s2r = s5r[:, 4:5]
    bcol = bcol_ref[...]
    br = sr.shape[0]
    e5 = jnp.where(
        jax.lax.broadcasted_iota(jnp.int32, (1, 5), 1) == 4, 1.0, 0.0)

    def d2_chunk(k, _):
        cols = pl.ds(k * _CC, _CC)
        s5c = s5_ref[cols, :]
        s2c = _dot_t(e5, s5c, precision=_HI)
        d2 = s2r + s2c - 2.0 * _dot_t(sr, s5c[:, 0:4])
        d2 = jnp.maximum(d2, 0.0)
        same = bcol == brow_ref[:, cols]
        d2 = jnp.where(same, d2, jnp.inf)
        bits_ref[:, cols] = jax.lax.bitcast_convert_type(d2, jnp.int32)
        return 0

    jax.lax.fori_loop(k_lo, k_hi, d2_chunk, 0)

    lo0 = jnp.full((br, 1), -1, jnp.int32)
    hi0 = jnp.full((br, 1), _INF_BITS, jnp.int32)

    def bs_cond(lh):
        lo, hi = lh
        return jnp.any(hi - lo > 1)

    def bs_body(lh):
        lo, hi = lh
        mid = lo + (hi - lo) // 2

        def cnt_chunk(k, acc):
            bits_c = bits_ref[:, pl.ds(k * _CC, _CC)]
            return acc + jnp.sum(jnp.where(bits_c <= mid, 1.0, 0.0),
                                 axis=1, keepdims=True)

        cnt = jax.lax.fori_loop(k_lo, k_hi, cnt_chunk, jnp.zeros((br, 1)))
        ge = cnt >= float(_K)
        hi = jnp.where(ge, mid, hi)
        lo = jnp.where(ge, lo, mid)
        lo = jnp.where(cnt == float(_K), hi - 1, lo)
        return lo, hi

    _, thr = jax.lax.while_loop(bs_cond, bs_body, (lo0, hi0))

    def agg_chunk(k, carry):
        mean_acc, max_acc = carry
        cols = pl.ds(k * _CC, _CC)
        bits_c = bits_ref[:, cols]
        d2 = jax.lax.bitcast_convert_type(bits_c, jnp.float32)
        sel = bits_c <= thr
        w = jnp.where(sel, jnp.exp(-10.0 * d2), 0.0)
        htc = ht_ref[:, cols]
        mean_acc = mean_acc + _dot_t(w, htc, precision=_HI)
        mx = []
        for dcol in range(ht_ref.shape[0]):
            vals = jnp.where(sel, w * htc[dcol:dcol + 1, :], -_BIG)
            mx.append(jnp.max(vals, axis=1, keepdims=True))
        return mean_acc, jnp.maximum(max_acc, jnp.concatenate(mx, axis=1))

    nh = ht_ref.shape[0]
    mean_agg, max_agg = jax.lax.fori_loop(
        k_lo, k_hi, agg_chunk,
        (jnp.zeros((br, nh)), jnp.full((br, nh), -_BIG)))
    mean_agg = mean_agg * (1.0 / _K)

    agg = jnp.concatenate([mean_agg, max_agg], axis=1)
    out0 = xo_ref[...] + _dot_t(agg, wo2_ref[...]) + bo2_ref[...]
    out0_ref[...] = out0

    @pl.when(pl.program_id(0) == 0)
    def _():
        sum1_ref[...] = jnp.zeros_like(sum1_ref)
        ssq1_ref[...] = jnp.zeros_like(ssq1_ref)

    sum1_ref[...] += jnp.sum(out0, axis=0, keepdims=True)
    ssq1_ref[...] += jnp.sum(out0 * out0, axis=0, keepdims=True)


def _make_bn_linear(n):
    def _stage(in_ref, sum_ref, ssq_ref, g_ref, be_ref, w_ref, b_ref,
               z_ref, sum2_ref, ssq2_ref):
        m = sum_ref[...] * (1.0 / n)
        v = ssq_ref[...] * (1.0 / n) - m * m
        xn = (in_ref[...] - m) * jax.lax.rsqrt(v + _EPS) * g_ref[...] + be_ref[...]
        z = jnp.tanh(_dot_t(xn, w_ref[...]) + b_ref[...])
        z_ref[...] = z

        @pl.when(pl.program_id(0) == 0)
        def _():
            sum2_ref[...] = jnp.zeros_like(sum2_ref)
            ssq2_ref[...] = jnp.zeros_like(ssq2_ref)

        sum2_ref[...] += jnp.sum(z, axis=0, keepdims=True)
        ssq2_ref[...] += jnp.sum(z * z, axis=0, keepdims=True)
    return _stage


def _make_c2(n):
    def _stage(z_ref, sum_ref, ssq_ref, g_ref, be_ref, w_ref, b_ref, bcol_ref,
               o2_ref, ssum_ref, smin_ref, smax_ref):
        m = sum_ref[...] * (1.0 / n)
        v = ssq_ref[...] * (1.0 / n) - m * m
        zn = (z_ref[...] - m) * jax.lax.rsqrt(v + _EPS) * g_ref[...] + be_ref[...]
        o2 = jnp.tanh(_dot_t(zn, w_ref[...]) + b_ref[...])
        o2_ref[...] = o2

        @pl.when(pl.program_id(0) == 0)
        def _():
            ssum_ref[...] = jnp.zeros_like(ssum_ref)
            smin_ref[...] = jnp.full_like(smin_ref, _BIG)
            smax_ref[...] = jnp.full_like(smax_ref, -_BIG)

        bcol = bcol_ref[...]
        for g in range(_NG):
            mg = bcol == float(g)
            ssum_ref[g:g + 1, :] += jnp.sum(
                jnp.where(mg, o2, 0.0), axis=0, keepdims=True)
            smin_ref[g:g + 1, :] = jnp.minimum(
                smin_ref[g:g + 1, :],
                jnp.min(jnp.where(mg, o2, _BIG), axis=0, keepdims=True))
            smax_ref[g:g + 1, :] = jnp.maximum(
                smax_ref[g:g + 1, :],
                jnp.max(jnp.where(mg, o2, -_BIG), axis=0, keepdims=True))
    return _stage


def _stage_c3(o2_ref, ssum_ref, smin_ref, smax_ref, bcol_ref, brow_ref,
              wout_ref, bout_ref, y_ref, sum3_ref, ssq3_ref):
    bcol = bcol_ref[...]
    brow = brow_ref[...]
    br = o2_ref.shape[0]
    mmm = jnp.zeros((br, 3 * 96), jnp.float32)
    for g in range(_NG):
        cnt = jnp.sum(jnp.where(brow == float(g), 1.0, 0.0),
                      axis=1, keepdims=True)
        cnt = jnp.maximum(cnt, 1.0)
        row = jnp.concatenate(
            [ssum_ref[g:g + 1, :] / cnt,
             smin_ref[g:g + 1, :],
             smax_ref[g:g + 1, :]], axis=1)
        mmm = mmm + jnp.where(bcol == float(g), 1.0, 0.0) * row
    cat = jnp.concatenate([mmm, o2_ref[...]], axis=1)
    y = jnp.tanh(_dot_t(cat, wout_ref[...]) + bout_ref[...])
    y_ref[...] = y

    @pl.when(pl.program_id(0) == 0)
    def _():
        sum3_ref[...] = jnp.zeros_like(sum3_ref)
        ssq3_ref[...] = jnp.zeros_like(ssq3_ref)

    sum3_ref[...] += jnp.sum(y, axis=0, keepdims=True)
    ssq3_ref[...] += jnp.sum(y * y, axis=0, keepdims=True)


def _make_c4(n):
    def _stage(y_ref, sum_ref, ssq_ref, g_ref, be_ref, out_ref):
        m = sum_ref[...] * (1.0 / n)
        v = ssq_ref[...] * (1.0 / n) - m * m
        out_ref[...] = ((y_ref[...] - m) * jax.lax.rsqrt(v + _EPS)
                        * g_ref[...] + be_ref[...])
    return _stage


def _whole(a):
    nd = a.ndim
    return pl.BlockSpec(a.shape, lambda i, *_, _n=nd: (0,) * _n)


def _rows(shape):
    return pl.BlockSpec(shape, lambda i, *_: (i, 0))


def _fixed(shape):
    return pl.BlockSpec(shape, lambda i, *_: (0, 0))


_SEQ = pltpu.CompilerParams(dimension_semantics=("arbitrary",))


def kernel(x, batch, params):
    p = params
    n, d_in = x.shape
    br = next(b for b in (80, 40, 16, 8, 1) if n % b == 0)
    nb = n // br
    f32 = jnp.float32
    grid = (nb,)

    bf = batch.astype(f32)
    bcol = bf.reshape(n, 1)
    brow = bf.reshape(1, n)

    bs = p['bs'].reshape(1, -1)
    bh = p['bh'].reshape(1, -1)
    bo2 = p['bo2'].reshape(1, -1)
    b1 = p['b1'].reshape(1, -1)
    b2 = p['b2'].reshape(1, -1)
    bout = p['bout'].reshape(1, -1)
    g1, be1 = p['g1'].reshape(1, -1), p['be1'].reshape(1, -1)
    g2, be2 = p['g2'].reshape(1, -1), p['be2'].reshape(1, -1)
    g3, be3 = p['g3'].reshape(1, -1), p['be3'].reshape(1, -1)

    s5, h, xo = pl.pallas_call(
        _stage_a,
        grid=grid,
        in_specs=[_rows((br, d_in)), _whole(p['Ws']), _whole(bs),
                  _whole(p['Wh']), _whole(bh), _whole(p['Wo1'])],
        out_specs=[_rows((br, 5)), _rows((br, 22)), _rows((br, 96))],
        out_shape=[jax.ShapeDtypeStruct((n, 5), f32),
                   jax.ShapeDtypeStruct((n, 22), f32),
                   jax.ShapeDtypeStruct((n, 96), f32)],
        compiler_params=_SEQ,
    )(x, p['Ws'], bs, p['Wh'], bh, p['Wo1'])

    ht = h.T

    n_pad = ((n + _CC - 1) // _CC) * _CC
    s5_p = jnp.pad(s5, ((0, n_pad - n), (0, 0)))
    ht_p = jnp.pad(ht, ((0, 0), (0, n_pad - n)))
    brow_p = jnp.pad(brow, ((0, 0), (0, n_pad - n)), constant_values=-1.0)
    bi = batch.astype(jnp.int32)
    first_g = bi[0::br]
    last_g = bi[br - 1::br]
    c_lo = jnp.searchsorted(bi, first_g, side='left')
    c_hi = jnp.searchsorted(bi, last_g, side='right')
    bnds = jnp.stack([c_lo // _CC,
                      (c_hi + _CC - 1) // _CC]).astype(jnp.int32)

    out0, sum1, ssq1 = pl.pallas_call(
        _stage_b,
        grid_spec=pltpu.PrefetchScalarGridSpec(
            num_scalar_prefetch=1,
            grid=grid,
            in_specs=[_rows((br, 5)), _rows((br, 1)), _whole(s5_p),
                      _whole(brow_p), _whole(ht_p), _rows((br, 96)),
                      _whole(p['Wo2']), _whole(bo2)],
            out_specs=[_rows((br, 96)), _fixed((1, 96)), _fixed((1, 96))],
            scratch_shapes=[pltpu.VMEM((br, n_pad), jnp.int32)],
        ),
        out_shape=[jax.ShapeDtypeStruct((n, 96), f32),
                   jax.ShapeDtypeStruct((1, 96), f32),
                   jax.ShapeDtypeStruct((1, 96), f32)],
        compiler_params=_SEQ,
    )(bnds, s5, bcol, s5_p, brow_p, ht_p, xo, p['Wo2'], bo2)

    z, sum2, ssq2 = pl.pallas_call(
        _make_bn_linear(float(n)),
        grid=grid,
        in_specs=[_rows((br, 96)), _fixed((1, 96)), _fixed((1, 96)),
                  _whole(g1), _whole(be1), _whole(p['W1']), _whole(b1)],
        out_specs=[_rows((br, 128)), _fixed((1, 128)), _fixed((1, 128))],
        out_shape=[jax.ShapeDtypeStruct((n, 128), f32),
                   jax.ShapeDtypeStruct((1, 128), f32),
                   jax.ShapeDtypeStruct((1, 128), f32)],
        compiler_params=_SEQ,
    )(out0, sum1, ssq1, g1, be1, p['W1'], b1)

    o2, ssum, smin, smax = pl.pallas_call(
        _make_c2(float(n)),
        grid=grid,
        in_specs=[_rows((br, 128)), _fixed((1, 128)), _fixed((1, 128)),
                  _whole(g2), _whole(be2), _whole(p['W2']), _whole(b2),
                  _rows((br, 1))],
        out_specs=[_rows((br, 96)), _fixed((_NG, 96)), _fixed((_NG, 96)),
                   _fixed((_NG, 96))],
        out_shape=[jax.ShapeDtypeStruct((n, 96), f32),
                   jax.ShapeDtypeStruct((_NG, 96), f32),
                   jax.ShapeDtypeStruct((_NG, 96), f32),
                   jax.ShapeDtypeStruct((_NG, 96), f32)],
        compiler_params=_SEQ,
    )(z, sum2, ssq2, g2, be2, p['W2'], b2, bcol)

    y, sum3, ssq3 = pl.pallas_call(
        _stage_c3,
        grid=grid,
        in_specs=[_rows((br, 96)), _fixed((_NG, 96)), _fixed((_NG, 96)),
                  _fixed((_NG, 96)), _rows((br, 1)), _whole(brow),
                  _whole(p['Wout']), _whole(bout)],
        out_specs=[_rows((br, 96)), _fixed((1, 96)), _fixed((1, 96))],
        out_shape=[jax.ShapeDtypeStruct((n, 96), f32),
                   jax.ShapeDtypeStruct((1, 96), f32),
                   jax.ShapeDtypeStruct((1, 96), f32)],
        compiler_params=_SEQ,
    )(o2, ssum, smin, smax, bcol, brow, p['Wout'], bout)

    (out,) = pl.pallas_call(
        _make_c4(float(n)),
        grid=grid,
        in_specs=[_rows((br, 96)), _fixed((1, 96)), _fixed((1, 96)),
                  _whole(g3), _whole(be3)],
        out_specs=[_rows((br, 96))],
        out_shape=[jax.ShapeDtypeStruct((n, 96), f32)],
        compiler_params=_SEQ,
    )(y, sum3, ssq3, g3, be3)

    return out


# speedup vs baseline: 1.7757x; 1.7757x over previous
import jax
import jax.numpy as jnp
from jax.experimental import pallas as pl
from jax.experimental.pallas import tpu as pltpu

_K = 40
_EPS = 1e-5
_NG = 8
_INF_BITS = 0x7F800000
_BIG = 3.0e38
_HI = jax.lax.Precision.HIGHEST


def _dot_t(a, b, precision=None):
    return jax.lax.dot_general(
        a, b, (((1,), (1,)), ((), ())),
        precision=precision, preferred_element_type=jnp.float32)


def _stage_a(x_ref, ws_ref, bs_ref, wh_ref, bh_ref, wo1_ref,
             s5_ref, h_ref, xo_ref):
    xb = x_ref[...]
    s = _dot_t(xb, ws_ref[...]) + bs_ref[...]
    s2 = jnp.sum(s * s, axis=1, keepdims=True)
    s5_ref[...] = jnp.concatenate([s, s2], axis=1)
    h_ref[...] = _dot_t(xb, wh_ref[...]) + bh_ref[...]
    xo_ref[...] = _dot_t(xb, wo1_ref[...])


_CC = 512


def _stage_b(bnds_ref, s5r_ref, bcol_ref, s5_ref, brow_ref, ht_ref, xo_ref,
             wo2_ref, bo2_ref, out0_ref, sum1_ref, ssq1_ref, bits_ref):
    i = pl.program_id(0)
    k_lo = bnds_ref[0, i]
    k_hi = bnds_ref[1, i]

    s5r = s5r_ref[...]
    sr = s5r[:, 0:4]
    s2r = s5r[:, 4:5]
    bcol = bcol_ref[...]
    br = sr.shape[0]
    e5 = jnp.where(
        jax.lax.broadcasted_iota(jnp.int32, (1, 5), 1) == 4, 1.0, 0.0)

    def d2_chunk(k, _):
        cols = pl.ds(k * _CC, _CC)
        s5c = s5_ref[cols, :]
        s2c = _dot_t(e5, s5c, precision=_HI)
        d2 = s2r + s2c - 2.0 * _dot_t(sr, s5c[:, 0:4])
        d2 = jnp.maximum(d2, 0.0)
        same = bcol == brow_ref[:, cols]
        d2 = jnp.where(same, d2, jnp.inf)
        bits_ref[:, cols] = jax.lax.bitcast_convert_type(d2, jnp.int32)
        return 0

    jax.lax.fori_loop(k_lo, k_hi, d2_chunk, 0)

    lo0 = jnp.full((br, 1), -1, jnp.int32)
    hi0 = jnp.full((br, 1), _INF_BITS, jnp.int32)

    def bs_body(_, lh):
        lo, hi = lh
        mid = lo + (hi - lo) // 2

        def cnt_chunk(k, acc):
            bits_c = bits_ref[:, pl.ds(k * _CC, _CC)]
            return acc + jnp.sum(jnp.where(bits_c <= mid, 1.0, 0.0),
                                 axis=1, keepdims=True)

        cnt = jax.lax.fori_loop(k_lo, k_hi, cnt_chunk, jnp.zeros((br, 1)))
        ge = cnt >= float(_K)
        return jnp.where(ge, lo, mid), jnp.where(ge, mid, hi)

    _, thr = jax.lax.fori_loop(0, 31, bs_body, (lo0, hi0))

    def agg_chunk(k, carry):
        mean_acc, max_acc = carry
        cols = pl.ds(k * _CC, _CC)
        bits_c = bits_ref[:, cols]
        d2 = jax.lax.bitcast_convert_type(bits_c, jnp.float32)
        sel = bits_c <= thr
        w = jnp.where(sel, jnp.exp(-10.0 * d2), 0.0)
        htc = ht_ref[:, cols]
        mean_acc = mean_acc + _dot_t(w, htc, precision=_HI)
        mx = []
        for dcol in range(ht_ref.shape[0]):
            vals = jnp.where(sel, w * htc[dcol:dcol + 1, :], -_BIG)
            mx.append(jnp.max(vals, axis=1, keepdims=True))
        return mean_acc, jnp.maximum(max_acc, jnp.concatenate(mx, axis=1))

    nh = ht_ref.shape[0]
    mean_agg, max_agg = jax.lax.fori_loop(
        k_lo, k_hi, agg_chunk,
        (jnp.zeros((br, nh)), jnp.full((br, nh), -_BIG)))
    mean_agg = mean_agg * (1.0 / _K)

    agg = jnp.concatenate([mean_agg, max_agg], axis=1)
    out0 = xo_ref[...] + _dot_t(agg, wo2_ref[...]) + bo2_ref[...]
    out0_ref[...] = out0

    @pl.when(pl.program_id(0) == 0)
    def _():
        sum1_ref[...] = jnp.zeros_like(sum1_ref)
        ssq1_ref[...] = jnp.zeros_like(ssq1_ref)

    sum1_ref[...] += jnp.sum(out0, axis=0, keepdims=True)
    ssq1_ref[...] += jnp.sum(out0 * out0, axis=0, keepdims=True)


def _make_bn_linear(n):
    def _stage(in_ref, sum_ref, ssq_ref, g_ref, be_ref, w_ref, b_ref,
               z_ref, sum2_ref, ssq2_ref):
        m = sum_ref[...] * (1.0 / n)
        v = ssq_ref[...] * (1.0 / n) - m * m
        xn = (in_ref[...] - m) * jax.lax.rsqrt(v + _EPS) * g_ref[...] + be_ref[...]
        z = jnp.tanh(_dot_t(xn, w_ref[...]) + b_ref[...])
        z_ref[...] = z

        @pl.when(pl.program_id(0) == 0)
        def _():
            sum2_ref[...] = jnp.zeros_like(sum2_ref)
            ssq2_ref[...] = jnp.zeros_like(ssq2_ref)

        sum2_ref[...] += jnp.sum(z, axis=0, keepdims=True)
        ssq2_ref[...] += jnp.sum(z * z, axis=0, keepdims=True)
    return _stage


def _make_c2(n):
    def _stage(z_ref, sum_ref, ssq_ref, g_ref, be_ref, w_ref, b_ref, bcol_ref,
               o2_ref, ssum_ref, smin_ref, smax_ref):
        m = sum_ref[...] * (1.0 / n)
        v = ssq_ref[...] * (1.0 / n) - m * m
        zn = (z_ref[...] - m) * jax.lax.rsqrt(v + _EPS) * g_ref[...] + be_ref[...]
        o2 = jnp.tanh(_dot_t(zn, w_ref[...]) + b_ref[...])
        o2_ref[...] = o2

        @pl.when(pl.program_id(0) == 0)
        def _():
            ssum_ref[...] = jnp.zeros_like(ssum_ref)
            smin_ref[...] = jnp.full_like(smin_ref, _BIG)
            smax_ref[...] = jnp.full_like(smax_ref, -_BIG)

        bcol = bcol_ref[...]
        for g in range(_NG):
            mg = bcol == float(g)
            ssum_ref[g:g + 1, :] += jnp.sum(
                jnp.where(mg, o2, 0.0), axis=0, keepdims=True)
            smin_ref[g:g + 1, :] = jnp.minimum(
                smin_ref[g:g + 1, :],
                jnp.min(jnp.where(mg, o2, _BIG), axis=0, keepdims=True))
            smax_ref[g:g + 1, :] = jnp.maximum(
                smax_ref[g:g + 1, :],
                jnp.max(jnp.where(mg, o2, -_BIG), axis=0, keepdims=True))
    return _stage


def _stage_c3(o2_ref, ssum_ref, smin_ref, smax_ref, bcol_ref, brow_ref,
              wout_ref, bout_ref, y_ref, sum3_ref, ssq3_ref):
    bcol = bcol_ref[...]
    brow = brow_ref[...]
    br = o2_ref.shape[0]
    mmm = jnp.zeros((br, 3 * 96), jnp.float32)
    for g in range(_NG):
        cnt = jnp.sum(jnp.where(brow == float(g), 1.0, 0.0),
                      axis=1, keepdims=True)
        cnt = jnp.maximum(cnt, 1.0)
        row = jnp.concatenate(
            [ssum_ref[g:g + 1, :] / cnt,
             smin_ref[g:g + 1, :],
             smax_ref[g:g + 1, :]], axis=1)
        mmm = mmm + jnp.where(bcol == float(g), 1.0, 0.0) * row
    cat = jnp.concatenate([mmm, o2_ref[...]], axis=1)
    y = jnp.tanh(_dot_t(cat, wout_ref[...]) + bout_ref[...])
    y_ref[...] = y

    @pl.when(pl.program_id(0) == 0)
    def _():
        sum3_ref[...] = jnp.zeros_like(sum3_ref)
        ssq3_ref[...] = jnp.zeros_like(ssq3_ref)

    sum3_ref[...] += jnp.sum(y, axis=0, keepdims=True)
    ssq3_ref[...] += jnp.sum(y * y, axis=0, keepdims=True)


def _make_c4(n):
    def _stage(y_ref, sum_ref, ssq_ref, g_ref, be_ref, out_ref):
        m = sum_ref[...] * (1.0 / n)
        v = ssq_ref[...] * (1.0 / n) - m * m
        out_ref[...] = ((y_ref[...] - m) * jax.lax.rsqrt(v + _EPS)
                        * g_ref[...] + be_ref[...])
    return _stage


def _whole(a):
    nd = a.ndim
    return pl.BlockSpec(a.shape, lambda i, *_, _n=nd: (0,) * _n)


def _rows(shape):
    return pl.BlockSpec(shape, lambda i, *_: (i, 0))


def _fixed(shape):
    return pl.BlockSpec(shape, lambda i, *_: (0, 0))


_SEQ = pltpu.CompilerParams(dimension_semantics=("arbitrary",))


def kernel(x, batch, params):
    p = params
    n, d_in = x.shape
    br = next(b for b in (200, 80, 40, 16, 8, 1) if n % b == 0)
    nb = n // br
    f32 = jnp.float32
    grid = (nb,)

    bf = batch.astype(f32)
    bcol = bf.reshape(n, 1)
    brow = bf.reshape(1, n)

    bs = p['bs'].reshape(1, -1)
    bh = p['bh'].reshape(1, -1)
    bo2 = p['bo2'].reshape(1, -1)
    b1 = p['b1'].reshape(1, -1)
    b2 = p['b2'].reshape(1, -1)
    bout = p['bout'].reshape(1, -1)
    g1, be1 = p['g1'].reshape(1, -1), p['be1'].reshape(1, -1)
    g2, be2 = p['g2'].reshape(1, -1), p['be2'].reshape(1, -1)
    g3, be3 = p['g3'].reshape(1, -1), p['be3'].reshape(1, -1)

    s5, h, xo = pl.pallas_call(
        _stage_a,
        grid=grid,
        in_specs=[_rows((br, d_in)), _whole(p['Ws']), _whole(bs),
                  _whole(p['Wh']), _whole(bh), _whole(p['Wo1'])],
        out_specs=[_rows((br, 5)), _rows((br, 22)), _rows((br, 96))],
        out_shape=[jax.ShapeDtypeStruct((n, 5), f32),
                   jax.ShapeDtypeStruct((n, 22), f32),
                   jax.ShapeDtypeStruct((n, 96), f32)],
        compiler_params=_SEQ,
    )(x, p['Ws'], bs, p['Wh'], bh, p['Wo1'])

    ht = h.T

    n_pad = ((n + _CC - 1) // _CC) * _CC
    s5_p = jnp.pad(s5, ((0, n_pad - n), (0, 0)))
    ht_p = jnp.pad(ht, ((0, 0), (0, n_pad - n)))
    brow_p = jnp.pad(brow, ((0, 0), (0, n_pad - n)), constant_values=-1.0)
    bi = batch.astype(jnp.int32)
    first_g = bi[0::br]
    last_g = bi[br - 1::br]
    c_lo = jnp.searchsorted(bi, first_g, side='left')
    c_hi = jnp.searchsorted(bi, last_g, side='right')
    bnds = jnp.stack([c_lo // _CC,
                      (c_hi + _CC - 1) // _CC]).astype(jnp.int32)

    out0, sum1, ssq1 = pl.pallas_call(
        _stage_b,
        grid_spec=pltpu.PrefetchScalarGridSpec(
            num_scalar_prefetch=1,
            grid=grid,
            in_specs=[_rows((br, 5)), _rows((br, 1)), _whole(s5_p),
                      _whole(brow_p), _whole(ht_p), _rows((br, 96)),
                      _whole(p['Wo2']), _whole(bo2)],
            out_specs=[_rows((br, 96)), _fixed((1, 96)), _fixed((1, 96))],
            scratch_shapes=[pltpu.VMEM((br, n_pad), jnp.int32)],
        ),
        out_shape=[jax.ShapeDtypeStruct((n, 96), f32),
                   jax.ShapeDtypeStruct((1, 96), f32),
                   jax.ShapeDtypeStruct((1, 96), f32)],
        compiler_params=_SEQ,
    )(bnds, s5, bcol, s5_p, brow_p, ht_p, xo, p['Wo2'], bo2)

    z, sum2, ssq2 = pl.pallas_call(
        _make_bn_linear(float(n)),
        grid=grid,
        in_specs=[_rows((br, 96)), _fixed((1, 96)), _fixed((1, 96)),
                  _whole(g1), _whole(be1), _whole(p['W1']), _whole(b1)],
        out_specs=[_rows((br, 128)), _fixed((1, 128)), _fixed((1, 128))],
        out_shape=[jax.ShapeDtypeStruct((n, 128), f32),
                   jax.ShapeDtypeStruct((1, 128), f32),
                   jax.ShapeDtypeStruct((1, 128), f32)],
        compiler_params=_SEQ,
    )(out0, sum1, ssq1, g1, be1, p['W1'], b1)

    o2, ssum, smin, smax = pl.pallas_call(
        _make_c2(float(n)),
        grid=grid,
        in_specs=[_rows((br, 128)), _fixed((1, 128)), _fixed((1, 128)),
                  _whole(g2), _whole(be2), _whole(p['W2']), _whole(b2),
                  _rows((br, 1))],
        out_specs=[_rows((br, 96)), _fixed((_NG, 96)), _fixed((_NG, 96)),
                   _fixed((_NG, 96))],
        out_shape=[jax.ShapeDtypeStruct((n, 96), f32),
                   jax.ShapeDtypeStruct((_NG, 96), f32),
                   jax.ShapeDtypeStruct((_NG, 96), f32),
                   jax.ShapeDtypeStruct((_NG, 96), f32)],
        compiler_params=_SEQ,
    )(z, sum2, ssq2, g2, be2, p['W2'], b2, bcol)

    y, sum3, ssq3 = pl.pallas_call(
        _stage_c3,
        grid=grid,
        in_specs=[_rows((br, 96)), _fixed((_NG, 96)), _fixed((_NG, 96)),
                  _fixed((_NG, 96)), _rows((br, 1)), _whole(brow),
                  _whole(p['Wout']), _whole(bout)],
        out_specs=[_rows((br, 96)), _fixed((1, 96)), _fixed((1, 96))],
        out_shape=[jax.ShapeDtypeStruct((n, 96), f32),
                   jax.ShapeDtypeStruct((1, 96), f32),
                   jax.ShapeDtypeStruct((1, 96), f32)],
        compiler_params=_SEQ,
    )(o2, ssum, smin, smax, bcol, brow, p['Wout'], bout)

    (out,) = pl.pallas_call(
        _make_c4(float(n)),
        grid=grid,
        in_specs=[_rows((br, 96)), _fixed((1, 96)), _fixed((1, 96)),
                  _whole(g3), _whole(be3)],
        out_specs=[_rows((br, 96))],
        out_shape=[jax.ShapeDtypeStruct((n, 96), f32)],
        compiler_params=_SEQ,
    )(y, sum3, ssq3, g3, be3)

    return out
